# Initial kernel scaffold; baseline (speedup 1.0000x reference)
import jax
import jax.numpy as jnp
from jax.experimental import pallas as pl


def kernel(X, vertex_ids, hyperedge_ids, W1, b1, W2, b2):
    raise NotImplementedError("write your pallas kernel here")


# speedup vs baseline: 7.5801x; 7.5801x over previous
import functools

import jax
import jax.numpy as jnp
from jax import lax
from jax.experimental import pallas as pl
from jax.experimental.pallas import tpu as pltpu
from jax.experimental.pallas import tpu_sc as plsc

N = 10000
NE = 5000
N_PAD = 10112
NE_PAD = 5120
M = 320000
NT = 32
PER_TILE = M // NT
CH = 125
NCH = PER_TILE // CH

_MESH = plsc.VectorSubcoreMesh(core_axis_name="c", subcore_axis_name="s")


def _make_phase(r_out):
  rows_per_tile = r_out // 16

  @functools.partial(
      pl.kernel,
      out_type=jax.ShapeDtypeStruct((2, r_out, 128), jnp.float32),
      mesh=_MESH,
      scratch_types=[
          pltpu.VMEM((NCH, CH), jnp.int32),
          pltpu.VMEM((NCH, CH), jnp.int32),
          pltpu.VMEM((CH, 128), jnp.float32),
          pltpu.VMEM_SHARED((r_out, 128), jnp.float32),
          pltpu.SemaphoreType.DMA,
      ],
  )
  def phase(table, gidx, sidx, zeros, out, gv, sv, buf, acc, sem):
    c = lax.axis_index("c")
    s = lax.axis_index("s")
    wid = c * 16 + s
    pltpu.sync_copy(gidx.at[wid], gv)
    pltpu.sync_copy(sidx.at[wid], sv)

    @pl.when(s == 0)
    def _():
      pltpu.sync_copy(zeros, acc)

    plsc.subcore_barrier()

    def body(j, carry):
      pltpu.async_copy(table.at[gv.at[j]], buf, sem).wait()
      pltpu.sync_copy(buf, acc.at[sv.at[j]], add=True)
      return carry

    lax.fori_loop(0, NCH, body, 0)
    plsc.subcore_barrier()
    r0 = s * rows_per_tile
    pltpu.sync_copy(acc.at[pl.ds(r0, rows_per_tile)],
                    out.at[c, pl.ds(r0, rows_per_tile)])

  return phase


_phase_n = _make_phase(N_PAD)
_phase_e = _make_phase(NE_PAD)


def _make_deg(r_out):
  rows_per_tile = r_out // 16

  @functools.partial(
      pl.kernel,
      out_type=jax.ShapeDtypeStruct((2, r_out, 128), jnp.float32),
      mesh=_MESH,
      scratch_types=[
          pltpu.VMEM((NCH, CH), jnp.int32),
          pltpu.VMEM((CH, 128), jnp.float32),
          pltpu.VMEM_SHARED((r_out, 128), jnp.float32),
      ],
  )
  def deg(sidx, ones, zeros, out, sv, ones_v, acc):
    c = lax.axis_index("c")
    s = lax.axis_index("s")
    wid = c * 16 + s
    pltpu.sync_copy(sidx.at[wid], sv)
    pltpu.sync_copy(ones, ones_v)

    @pl.when(s == 0)
    def _():
      pltpu.sync_copy(zeros, acc)

    plsc.subcore_barrier()

    def body(j, carry):
      pltpu.sync_copy(ones_v, acc.at[sv.at[j]], add=True)
      return carry

    lax.fori_loop(0, NCH, body, 0)
    plsc.subcore_barrier()
    r0 = s * rows_per_tile
    pltpu.sync_copy(acc.at[pl.ds(r0, rows_per_tile)],
                    out.at[c, pl.ds(r0, rows_per_tile)])

  return deg


_deg_n = _make_deg(N_PAD)
_deg_e = _make_deg(NE_PAD)


def _mm1_body(x_ref, w_ref, b_ref, dv_ref, de_ref, t1s_ref, dvis_ref,
              dei_ref):
  deg_v = dv_ref[0, 0:N, 0:1] + dv_ref[1, 0:N, 0:1]
  dvis = jnp.where(deg_v > 0, lax.rsqrt(deg_v), 0.0)
  deg_e = de_ref[0, :, 0:1] + de_ref[1, :, 0:1]
  dei_ref[...] = jnp.where(deg_e > 0, 1.0 / deg_e, 0.0)
  t1 = jnp.dot(x_ref[...], w_ref[...],
               preferred_element_type=jnp.float32) + b_ref[...]
  t1s_ref[...] = t1 * dvis
  dvis_ref[...] = dvis


def _comb_body(pe_ref, dei_ref, ef_ref):
  ef_ref[...] = (pe_ref[0] + pe_ref[1]) * dei_ref[...]


def _mm2_body(pv_ref, dvis_ref, w_ref, b_ref, t2s_ref):
  h = jax.nn.relu((pv_ref[0, 0:N, :] + pv_ref[1, 0:N, :]) * dvis_ref[...])
  t2 = (jnp.dot(h, w_ref[...], preferred_element_type=jnp.float32)
        + b_ref[...]) * dvis_ref[...]
  t2s_ref[...] = jnp.concatenate([t2, jnp.zeros_like(t2)], axis=1)


def _out_body(pv_ref, dvis_ref, out_ref):
  out_ref[...] = (pv_ref[0, 0:N, 0:64] + pv_ref[1, 0:N, 0:64]) * dvis_ref[...]


def kernel(X, vertex_ids, hyperedge_ids, W1, b1, W2, b2):
  vids3 = vertex_ids.reshape(NT, NCH, CH)
  eids3 = hyperedge_ids.reshape(NT, NCH, CH)
  ones = jnp.ones((CH, 128), jnp.float32)
  zn128 = jnp.zeros((N_PAD, 128), jnp.float32)
  ze128 = jnp.zeros((NE_PAD, 128), jnp.float32)

  pv_deg = _deg_n(vids3, ones, zn128)
  pe_deg = _deg_e(eids3, ones, ze128)

  t1s, dvis, dei = pl.pallas_call(
      _mm1_body,
      out_shape=(jax.ShapeDtypeStruct((N, 128), jnp.float32),
                 jax.ShapeDtypeStruct((N, 1), jnp.float32),
                 jax.ShapeDtypeStruct((NE_PAD, 1), jnp.float32)),
  )(X, W1, b1.reshape(1, 128), pv_deg, pe_deg)

  pe1 = _phase_e(t1s, vids3, eids3, ze128)
  ef1 = pl.pallas_call(
      _comb_body,
      out_shape=jax.ShapeDtypeStruct((NE_PAD, 128), jnp.float32),
  )(pe1, dei)

  pv1 = _phase_n(ef1, eids3, vids3, zn128)
  t2s = pl.pallas_call(
      _mm2_body,
      out_shape=jax.ShapeDtypeStruct((N, 128), jnp.float32),
  )(pv1, dvis, W2, b2.reshape(1, 64))

  pe2 = _phase_e(t2s, vids3, eids3, ze128)
  ef2 = pl.pallas_call(
      _comb_body,
      out_shape=jax.ShapeDtypeStruct((NE_PAD, 128), jnp.float32),
  )(pe2, dei)

  pv2 = _phase_n(ef2, eids3, vids3, zn128)
  out = pl.pallas_call(
      _out_body,
      out_shape=jax.ShapeDtypeStruct((N, 64), jnp.float32),
  )(pv2, dvis)
  return out


# speedup vs baseline: 8.9534x; 1.1812x over previous
import functools

import jax
import jax.numpy as jnp
from jax import lax
from jax.experimental import pallas as pl
from jax.experimental.pallas import tpu as pltpu
from jax.experimental.pallas import tpu_sc as plsc

N = 10000
NE = 5000
N_PAD = 10112
NE_PAD = 5120
M = 320000
NT = 32
PER_TILE = M // NT
CH = 125
NCH = PER_TILE // CH

_MESH = plsc.VectorSubcoreMesh(core_axis_name="c", subcore_axis_name="s")


def _make_phase(r_out, dbuf):
  rows_per_tile = r_out // 16

  @functools.partial(
      pl.kernel,
      out_type=jax.ShapeDtypeStruct((2, r_out, 128), jnp.float32),
      mesh=_MESH,
      scratch_types=[
          pltpu.VMEM((2, NCH, CH), jnp.int32),
          pltpu.VMEM((2 if dbuf else 1, CH, 128), jnp.float32),
          pltpu.VMEM_SHARED((r_out, 128), jnp.float32),
          pltpu.SemaphoreType.DMA,
      ],
  )
  def phase(table, gidx, sidx, zeros, out, idx, bufs, acc, sem):
    c = lax.axis_index("c")
    s = lax.axis_index("s")
    wid = c * 16 + s
    gv = idx.at[0]
    sv = idx.at[1]
    buf0 = bufs.at[0]
    buf1 = bufs.at[1] if dbuf else bufs.at[0]
    pltpu.sync_copy(gidx.at[wid], gv)
    pltpu.sync_copy(sidx.at[wid], sv)

    @pl.when(s == 0)
    def _():
      pltpu.sync_copy(zeros, acc)

    plsc.subcore_barrier()

    if dbuf:
      pltpu.async_copy(table.at[gv.at[0]], buf0, sem)

      def body(j2, carry):
        j = 2 * j2
        pltpu.async_copy(table.at[gv.at[j + 1]], buf1, sem)
        pltpu.make_async_copy(table.at[gv.at[j]], buf0, sem).wait()
        pltpu.sync_copy(buf0, acc.at[sv.at[j]], add=True)

        @pl.when(j + 2 < NCH)
        def _():
          pltpu.async_copy(table.at[gv.at[j + 2]], buf0, sem)

        pltpu.make_async_copy(table.at[gv.at[j + 1]], buf1, sem).wait()
        pltpu.sync_copy(buf1, acc.at[sv.at[j + 1]], add=True)
        return carry

      lax.fori_loop(0, NCH // 2, body, 0)
    else:
      def body(j, carry):
        pltpu.async_copy(table.at[gv.at[j]], buf0, sem).wait()
        pltpu.sync_copy(buf0, acc.at[sv.at[j]], add=True)
        return carry

      lax.fori_loop(0, NCH, body, 0)
    plsc.subcore_barrier()
    r0 = s * rows_per_tile
    pltpu.sync_copy(acc.at[pl.ds(r0, rows_per_tile)],
                    out.at[c, pl.ds(r0, rows_per_tile)])

  return phase


_phase_n = _make_phase(N_PAD, dbuf=False)
_phase_e = _make_phase(NE_PAD, dbuf=True)


def _make_deg(r_out):
  rows_per_tile = r_out // 16

  @functools.partial(
      pl.kernel,
      out_type=jax.ShapeDtypeStruct((2, r_out, 128), jnp.float32),
      mesh=_MESH,
      scratch_types=[
          pltpu.VMEM((NCH, CH), jnp.int32),
          pltpu.VMEM((CH, 128), jnp.float32),
          pltpu.VMEM_SHARED((r_out, 128), jnp.float32),
      ],
  )
  def deg(sidx, ones, zeros, out, sv, ones_v, acc):
    c = lax.axis_index("c")
    s = lax.axis_index("s")
    wid = c * 16 + s
    pltpu.sync_copy(sidx.at[wid], sv)
    pltpu.sync_copy(ones, ones_v)

    @pl.when(s == 0)
    def _():
      pltpu.sync_copy(zeros, acc)

    plsc.subcore_barrier()

    def body(j, carry):
      pltpu.sync_copy(ones_v, acc.at[sv.at[j]], add=True)
      return carry

    lax.fori_loop(0, NCH, body, 0)
    plsc.subcore_barrier()
    r0 = s * rows_per_tile
    pltpu.sync_copy(acc.at[pl.ds(r0, rows_per_tile)],
                    out.at[c, pl.ds(r0, rows_per_tile)])

  return deg


_deg_n = _make_deg(N_PAD)
_deg_e = _make_deg(NE_PAD)


def _mm1_body(x_ref, w_ref, b_ref, dv_ref, de_ref, t1s_ref, dvis_ref,
              dei_ref):
  deg_v = dv_ref[0, 0:N, 0:1] + dv_ref[1, 0:N, 0:1]
  dvis = jnp.where(deg_v > 0, lax.rsqrt(deg_v), 0.0)
  deg_e = de_ref[0, :, 0:1] + de_ref[1, :, 0:1]
  dei_ref[...] = jnp.where(deg_e > 0, 1.0 / deg_e, 0.0)
  t1 = jnp.dot(x_ref[...], w_ref[...],
               preferred_element_type=jnp.float32) + b_ref[...]
  t1s_ref[...] = t1 * dvis
  dvis_ref[...] = dvis


def _comb_body(pe_ref, dei_ref, ef_ref):
  ef_ref[...] = (pe_ref[0] + pe_ref[1]) * dei_ref[...]


def _mm2_body(pv_ref, dvis_ref, w_ref, b_ref, t2s_ref):
  h = jax.nn.relu((pv_ref[0, 0:N, :] + pv_ref[1, 0:N, :]) * dvis_ref[...])
  t2 = (jnp.dot(h, w_ref[...], preferred_element_type=jnp.float32)
        + b_ref[...]) * dvis_ref[...]
  t2s_ref[...] = jnp.concatenate([t2, jnp.zeros_like(t2)], axis=1)


def _out_body(pv_ref, dvis_ref, out_ref):
  out_ref[...] = (pv_ref[0, 0:N, 0:64] + pv_ref[1, 0:N, 0:64]) * dvis_ref[...]


def kernel(X, vertex_ids, hyperedge_ids, W1, b1, W2, b2):
  vids3 = vertex_ids.reshape(NT, NCH, CH)
  eids3 = hyperedge_ids.reshape(NT, NCH, CH)
  ones = jnp.ones((CH, 128), jnp.float32)
  zn128 = jnp.zeros((N_PAD, 128), jnp.float32)
  ze128 = jnp.zeros((NE_PAD, 128), jnp.float32)

  pv_deg = _deg_n(vids3, ones, zn128)
  pe_deg = _deg_e(eids3, ones, ze128)

  t1s, dvis, dei = pl.pallas_call(
      _mm1_body,
      out_shape=(jax.ShapeDtypeStruct((N, 128), jnp.float32),
                 jax.ShapeDtypeStruct((N, 1), jnp.float32),
                 jax.ShapeDtypeStruct((NE_PAD, 1), jnp.float32)),
  )(X, W1, b1.reshape(1, 128), pv_deg, pe_deg)

  pe1 = _phase_e(t1s, vids3, eids3, ze128)
  ef1 = pl.pallas_call(
      _comb_body,
      out_shape=jax.ShapeDtypeStruct((NE_PAD, 128), jnp.float32),
  )(pe1, dei)

  pv1 = _phase_n(ef1, eids3, vids3, zn128)
  t2s = pl.pallas_call(
      _mm2_body,
      out_shape=jax.ShapeDtypeStruct((N, 128), jnp.float32),
  )(pv1, dvis, W2, b2.reshape(1, 64))

  pe2 = _phase_e(t2s, vids3, eids3, ze128)
  ef2 = pl.pallas_call(
      _comb_body,
      out_shape=jax.ShapeDtypeStruct((NE_PAD, 128), jnp.float32),
  )(pe2, dei)

  pv2 = _phase_n(ef2, eids3, vids3, zn128)
  out = pl.pallas_call(
      _out_body,
      out_shape=jax.ShapeDtypeStruct((N, 64), jnp.float32),
  )(pv2, dvis)
  return out


# speedup vs baseline: 9.7097x; 1.0845x over previous
import functools

import jax
import jax.numpy as jnp
from jax import lax
from jax.experimental import pallas as pl
from jax.experimental.pallas import tpu as pltpu
from jax.experimental.pallas import tpu_sc as plsc

N = 10000
NE = 5000
N_PAD = 10112
NE_PAD = 5120
M = 320000
NT = 32
PER_TILE = M // NT
CH = 125
NCH = PER_TILE // CH

_MESH = plsc.VectorSubcoreMesh(core_axis_name="c", subcore_axis_name="s")


def _make_phase(r_out, dbuf):
  rows_per_tile = r_out // 16

  @functools.partial(
      pl.kernel,
      out_type=jax.ShapeDtypeStruct((2, r_out, 128), jnp.float32),
      mesh=_MESH,
      scratch_types=[
          pltpu.VMEM((2, NCH, CH), jnp.int32),
          pltpu.VMEM((2 if dbuf else 1, CH, 128), jnp.float32),
          pltpu.VMEM_SHARED((r_out, 128), jnp.float32),
          pltpu.SemaphoreType.DMA,
      ],
  )
  def phase(table, gidx, sidx, zeros, out, idx, bufs, acc, sem):
    c = lax.axis_index("c")
    s = lax.axis_index("s")
    wid = c * 16 + s
    gv = idx.at[0]
    sv = idx.at[1]
    buf0 = bufs.at[0]
    buf1 = bufs.at[1] if dbuf else bufs.at[0]
    pltpu.sync_copy(gidx.at[wid], gv)
    pltpu.sync_copy(sidx.at[wid], sv)

    @pl.when(s == 0)
    def _():
      pltpu.sync_copy(zeros, acc)

    plsc.subcore_barrier()

    if dbuf:
      pltpu.async_copy(table.at[gv.at[0]], buf0, sem)

      def body(j2, carry):
        j = 2 * j2
        pltpu.async_copy(table.at[gv.at[j + 1]], buf1, sem)
        pltpu.make_async_copy(table.at[gv.at[j]], buf0, sem).wait()
        pltpu.sync_copy(buf0, acc.at[sv.at[j]], add=True)

        @pl.when(j + 2 < NCH)
        def _():
          pltpu.async_copy(table.at[gv.at[j + 2]], buf0, sem)

        pltpu.make_async_copy(table.at[gv.at[j + 1]], buf1, sem).wait()
        pltpu.sync_copy(buf1, acc.at[sv.at[j + 1]], add=True)
        return carry

      lax.fori_loop(0, NCH // 2, body, 0)
    else:
      def body(j, carry):
        pltpu.async_copy(table.at[gv.at[j]], buf0, sem).wait()
        pltpu.sync_copy(buf0, acc.at[sv.at[j]], add=True)
        return carry

      lax.fori_loop(0, NCH, body, 0)
    plsc.subcore_barrier()
    r0 = s * rows_per_tile
    pltpu.sync_copy(acc.at[pl.ds(r0, rows_per_tile)],
                    out.at[c, pl.ds(r0, rows_per_tile)])

  return phase


_phase_n = _make_phase(N_PAD, dbuf=False)
_phase_e = _make_phase(NE_PAD, dbuf=True)
N2_PAD = 5120
_phase_n2 = _make_phase(N2_PAD, dbuf=True)


def _make_deg(r_out):
  rows_per_tile = r_out // 16

  @functools.partial(
      pl.kernel,
      out_type=jax.ShapeDtypeStruct((2, r_out, 128), jnp.float32),
      mesh=_MESH,
      scratch_types=[
          pltpu.VMEM((NCH, CH), jnp.int32),
          pltpu.VMEM((CH, 128), jnp.float32),
          pltpu.VMEM_SHARED((r_out, 128), jnp.float32),
      ],
  )
  def deg(sidx, ones, zeros, out, sv, ones_v, acc):
    c = lax.axis_index("c")
    s = lax.axis_index("s")
    wid = c * 16 + s
    pltpu.sync_copy(sidx.at[wid], sv)
    pltpu.sync_copy(ones, ones_v)

    @pl.when(s == 0)
    def _():
      pltpu.sync_copy(zeros, acc)

    plsc.subcore_barrier()

    def body(j, carry):
      pltpu.sync_copy(ones_v, acc.at[sv.at[j]], add=True)
      return carry

    lax.fori_loop(0, NCH, body, 0)
    plsc.subcore_barrier()
    r0 = s * rows_per_tile
    pltpu.sync_copy(acc.at[pl.ds(r0, rows_per_tile)],
                    out.at[c, pl.ds(r0, rows_per_tile)])

  return deg


_deg_n = _make_deg(N_PAD)
_deg_e = _make_deg(NE_PAD)


def _mm1_body(x_ref, w_ref, b_ref, dv_ref, de_ref, t1s_ref, dvis_ref,
              dei_ref):
  deg_v = dv_ref[0, 0:N, 0:1] + dv_ref[1, 0:N, 0:1]
  dvis = jnp.where(deg_v > 0, lax.rsqrt(deg_v), 0.0)
  deg_e = de_ref[0, :, 0:1] + de_ref[1, :, 0:1]
  dei_ref[...] = jnp.where(deg_e > 0, 1.0 / deg_e, 0.0)
  t1 = jnp.dot(x_ref[...], w_ref[...],
               preferred_element_type=jnp.float32) + b_ref[...]
  t1s_ref[...] = t1 * dvis
  dvis_ref[...] = dvis


def _comb_body(pe_ref, dei_ref, ef_ref):
  ef_ref[...] = (pe_ref[0] + pe_ref[1]) * dei_ref[...]


def _comb2_body(pe_ref, dei_ref, ef2_ref):
  ef = (pe_ref[0] + pe_ref[1]) * dei_ref[...]
  lo = ef[:, 0:64]
  z = jnp.zeros_like(lo)
  ef2_ref[...] = jnp.concatenate([lo, z, z, lo], axis=1)


def _mm2_body(pv_ref, dvis_ref, w_ref, b_ref, t2s_ref):
  h = jax.nn.relu((pv_ref[0, 0:N, :] + pv_ref[1, 0:N, :]) * dvis_ref[...])
  t2 = (jnp.dot(h, w_ref[...], preferred_element_type=jnp.float32)
        + b_ref[...]) * dvis_ref[...]
  t2s_ref[...] = jnp.concatenate([t2, jnp.zeros_like(t2)], axis=1)


def _out_body(pv_ref, dvis_ref, out_ref):
  out_ref[...] = (pv_ref[0, 0:N, :] + pv_ref[1, 0:N, :]) * dvis_ref[...]


def kernel(X, vertex_ids, hyperedge_ids, W1, b1, W2, b2):
  vids3 = vertex_ids.reshape(NT, NCH, CH)
  eids3 = hyperedge_ids.reshape(NT, NCH, CH)
  ones = jnp.ones((CH, 128), jnp.float32)
  zn128 = jnp.zeros((N_PAD, 128), jnp.float32)
  ze128 = jnp.zeros((NE_PAD, 128), jnp.float32)

  pv_deg = _deg_n(vids3, ones, zn128)
  pe_deg = _deg_e(eids3, ones, ze128)

  t1s, dvis, dei = pl.pallas_call(
      _mm1_body,
      out_shape=(jax.ShapeDtypeStruct((N, 128), jnp.float32),
                 jax.ShapeDtypeStruct((N, 1), jnp.float32),
                 jax.ShapeDtypeStruct((NE_PAD, 1), jnp.float32)),
  )(X, W1, b1.reshape(1, 128), pv_deg, pe_deg)

  pe1 = _phase_e(t1s, vids3, eids3, ze128)
  ef1 = pl.pallas_call(
      _comb_body,
      out_shape=jax.ShapeDtypeStruct((NE_PAD, 128), jnp.float32),
  )(pe1, dei)

  pv1 = _phase_n(ef1, eids3, vids3, zn128)
  t2s = pl.pallas_call(
      _mm2_body,
      out_shape=jax.ShapeDtypeStruct((N, 128), jnp.float32),
  )(pv1, dvis, W2, b2.reshape(1, 64))

  pe2 = _phase_e(t2s, vids3, eids3, ze128)
  ef2 = pl.pallas_call(
      _comb2_body,
      out_shape=jax.ShapeDtypeStruct((NE_PAD, 256), jnp.float32),
  )(pe2, dei).reshape(2 * NE_PAD, 128)

  gidx2 = (2 * hyperedge_ids + (vertex_ids & 1)).reshape(NT, NCH, CH)
  sidx2 = (vertex_ids >> 1).reshape(NT, NCH, CH)
  zn2 = jnp.zeros((N2_PAD, 128), jnp.float32)
  pv2 = _phase_n2(ef2, gidx2, sidx2, zn2).reshape(2, 2 * N2_PAD, 64)
  out = pl.pallas_call(
      _out_body,
      out_shape=jax.ShapeDtypeStruct((N, 64), jnp.float32),
  )(pv2, dvis)
  return out
